# baseline (speedup 1.0000x reference)
import functools

import jax
import jax.numpy as jnp
from jax import lax
from jax.experimental import pallas as pl
from jax.experimental.pallas import tpu as pltpu
from jax.experimental.pallas import tpu_sc as plsc

N = 100000
D = 512
S = 256
H1, H2 = 256, 128

T = 2000

RW = 6272
CH = 128
CW = 256


def _mlp_body(x_ref, seg_ref, w1_ref, b1_ref, w2_ref, b2_ref, w3_ref, b3_ref,
              a_ref, den_ref):
    i = pl.program_id(0)
    x = x_ref[...]
    h = jnp.maximum(
        jnp.dot(x, w1_ref[...], preferred_element_type=jnp.float32)
        + b1_ref[...], 0.0)
    h = jnp.maximum(
        jnp.dot(h, w2_ref[...], preferred_element_type=jnp.float32)
        + b2_ref[...], 0.0)
    logit = jnp.sum(h * w3_ref[...], axis=1, keepdims=True) + b3_ref[...]
    a = jnp.exp(logit)
    a_ref[...] = jnp.broadcast_to(a, (a.shape[0], 16))

    seg = seg_ref[0, 0, :]
    onehot = (seg[None, :] == lax.broadcasted_iota(jnp.int32, (S, T), 0)
              ).astype(jnp.float32)
    a_b = jnp.broadcast_to(a, (a.shape[0], 128))

    @pl.when(i == 0)
    def _init():
        den_ref[...] = jnp.zeros_like(den_ref)

    den_ref[...] += jnp.dot(onehot, a_b, preferred_element_type=jnp.float32)


def _mlp_call(x, seg, W1, b1, W2, b2, W3, b3):
    grid = (N // T,)
    return pl.pallas_call(
        _mlp_body,
        grid=grid,
        in_specs=[
            pl.BlockSpec((T, D), lambda i: (i, 0)),
            pl.BlockSpec((1, 1, T), lambda i: (i, 0, 0)),
            pl.BlockSpec((D, H1), lambda i: (0, 0)),
            pl.BlockSpec((1, H1), lambda i: (0, 0)),
            pl.BlockSpec((H1, H2), lambda i: (0, 0)),
            pl.BlockSpec((1, H2), lambda i: (0, 0)),
            pl.BlockSpec((1, H2), lambda i: (0, 0)),
            pl.BlockSpec((1, 1), lambda i: (0, 0)),
        ],
        out_specs=[
            pl.BlockSpec((T, 16), lambda i: (i, 0)),
            pl.BlockSpec((S, 128), lambda i: (0, 0)),
        ],
        out_shape=[
            jax.ShapeDtypeStruct((N, 16), jnp.float32),
            jax.ShapeDtypeStruct((S, 128), jnp.float32),
        ],
    )(x, seg.reshape(N // T, 1, T), W1, b1.reshape(1, H1), W2,
      b2.reshape(1, H2), W3.reshape(1, H2), b3.reshape(1, 1))


def _seg_body(x_hbm, a_hbm, seg_hbm, pn_hbm, xbuf, abuf, segbuf, acc):
    c = lax.axis_index("c")
    s = lax.axis_index("s")
    base = s * RW
    nfull = lax.select(s == 15, (N - 15 * RW) // CH, RW // CH)
    ntail = lax.select(s == 15, (N - 15 * RW - ((N - 15 * RW) // CH) * CH) // 8,
                       0)
    col0 = c * CW
    iota = lax.broadcasted_iota(jnp.int32, (16,), 0)

    def zbody(k, carry):
        acc[k // (CW // 16), pl.ds((k % (CW // 16)) * 16, 16)] = jnp.zeros(
            (16,), jnp.float32)
        return carry

    lax.fori_loop(0, S * CW // 16, zbody, 0)

    NJ = CW // 16
    zero16 = jnp.zeros((16,), jnp.float32)

    def row_block(nrows):
        def seg_at(r):
            rvec = jnp.full((16,), r, jnp.int32)
            return jnp.max(plsc.load_gather(segbuf, [rvec]))

        def flush(prev, cregs):
            for j in range(NJ):
                plsc.addupdate(acc.at[prev, pl.ds(16 * j, 16)], cregs[j])
            return (zero16,) * NJ

        def rbody(r, carry):
            prev, cregs = carry
            seg_r = seg_at(r)
            cregs = lax.cond(seg_r != prev,
                             lambda cr: flush(prev, cr),
                             lambda cr: cr, cregs)
            av = abuf[r, :]
            cregs = tuple(cregs[j] + xbuf[r, pl.ds(16 * j, 16)] * av
                          for j in range(NJ))
            return (seg_r, cregs)

        def gbody(gi, carry):
            prev, cregs = carry
            r0 = gi * 16
            va = segbuf[pl.ds(r0, 16)]
            uniform = (jnp.max(va) == prev) & (jnp.min(va) == prev)

            def fast(carry_in):
                _, cr = carry_in
                for rr in range(16):
                    av = abuf[r0 + rr, :]
                    cr = tuple(cr[j] + xbuf[r0 + rr, pl.ds(16 * j, 16)] * av
                               for j in range(NJ))
                return (prev, cr)

            def slow(carry_in):
                return lax.fori_loop(r0, r0 + 16, rbody, carry_in)

            return lax.cond(uniform, fast, slow, (prev, cregs))

        prev0 = seg_at(0)
        init = (prev0, (zero16,) * NJ)
        if nrows % 16 == 0:
            prev, cregs = lax.fori_loop(0, nrows // 16, gbody, init)
        else:
            prev, cregs = lax.fori_loop(0, nrows, rbody, init)
        flush(prev, cregs)

    def body(i, carry):
        off = base + i * CH
        pltpu.sync_copy(x_hbm.at[pl.ds(off, CH), pl.ds(col0, CW)], xbuf)
        pltpu.sync_copy(a_hbm.at[pl.ds(off, CH)], abuf)
        pltpu.sync_copy(seg_hbm.at[pl.ds(off, CH)], segbuf)
        row_block(CH)
        return carry

    lax.fori_loop(0, nfull, body, 0)

    def tbody(i, carry):
        off = base + nfull * CH + i * 8
        pltpu.sync_copy(x_hbm.at[pl.ds(off, 8), pl.ds(col0, CW)],
                        xbuf.at[pl.ds(0, 8)])
        pltpu.sync_copy(a_hbm.at[pl.ds(off, 8)], abuf.at[pl.ds(0, 8)])
        pltpu.sync_copy(seg_hbm.at[pl.ds(off, 8)], segbuf.at[pl.ds(0, 8)])
        row_block(8)
        return carry

    lax.fori_loop(0, ntail, tbody, 0)

    pltpu.sync_copy(acc, pn_hbm.at[s, :, pl.ds(col0, CW)])


def _seg_call(x, a2, seg):
    mesh = plsc.VectorSubcoreMesh(core_axis_name="c", subcore_axis_name="s")
    f = pl.kernel(
        _seg_body,
        out_type=jax.ShapeDtypeStruct((16, S, D), jnp.float32),
        mesh=mesh,
        compiler_params=pltpu.CompilerParams(needs_layout_passes=False),
        scratch_types=[
            pltpu.VMEM((CH, CW), jnp.float32),
            pltpu.VMEM((CH, 16), jnp.float32),
            pltpu.VMEM((CH,), jnp.int32),
            pltpu.VMEM((S, CW), jnp.float32),
        ],
    )
    return f(x, a2, seg)


def _fin_body(pn_ref, den_ref, o_ref):
    i = pl.program_id(0)

    @pl.when(i == 0)
    def _init():
        o_ref[...] = jnp.zeros_like(o_ref)

    o_ref[...] += pn_ref[0]

    @pl.when(i == 15)
    def _done():
        o_ref[...] = o_ref[...] / den_ref[:, 0:1]


def _fin_call(pn, den):
    return pl.pallas_call(
        _fin_body,
        grid=(16,),
        in_specs=[
            pl.BlockSpec((1, S, D), lambda i: (i, 0, 0)),
            pl.BlockSpec((S, 128), lambda i: (0, 0)),
        ],
        out_specs=pl.BlockSpec((S, D), lambda i: (0, 0)),
        out_shape=jax.ShapeDtypeStruct((S, D), jnp.float32),
    )(pn, den)


def kernel(x, batch_index, W1, b1, W2, b2, W3, b3):
    seg = batch_index.astype(jnp.int32)
    a2, den = _mlp_call(x, seg, W1, b1, W2, b2, W3, b3)
    pn = _seg_call(x, a2, seg)
    return _fin_call(pn, den)


# speedup vs baseline: 1.1704x; 1.1704x over previous
import functools

import jax
import jax.numpy as jnp
from jax import lax
from jax.experimental import pallas as pl
from jax.experimental.pallas import tpu as pltpu
from jax.experimental.pallas import tpu_sc as plsc

N = 100000
D = 512
S = 256
H1, H2 = 256, 128

T = 2000

RW = 6272
CH = 128
CW = 256


def _mlp_body(x_ref, seg_ref, w1_ref, b1_ref, w2_ref, b2_ref, w3_ref, b3_ref,
              y_ref, den_ref):
    i = pl.program_id(0)
    x = x_ref[...]
    h = jnp.maximum(
        jnp.dot(x, w1_ref[...], preferred_element_type=jnp.float32)
        + b1_ref[...], 0.0)
    h = jnp.maximum(
        jnp.dot(h, w2_ref[...], preferred_element_type=jnp.float32)
        + b2_ref[...], 0.0)
    logit = jnp.sum(h * w3_ref[...], axis=1, keepdims=True) + b3_ref[...]
    a = jnp.exp(logit)
    y_ref[...] = x * a

    seg = seg_ref[0, 0, :]
    onehot = (seg[None, :] == lax.broadcasted_iota(jnp.int32, (S, T), 0)
              ).astype(jnp.float32)
    a_b = jnp.broadcast_to(a, (a.shape[0], 128))

    @pl.when(i == 0)
    def _init():
        den_ref[...] = jnp.zeros_like(den_ref)

    den_ref[...] += jnp.dot(onehot, a_b, preferred_element_type=jnp.float32)


def _mlp_call(x, seg, W1, b1, W2, b2, W3, b3):
    grid = (N // T,)
    return pl.pallas_call(
        _mlp_body,
        grid=grid,
        in_specs=[
            pl.BlockSpec((T, D), lambda i: (i, 0)),
            pl.BlockSpec((1, 1, T), lambda i: (i, 0, 0)),
            pl.BlockSpec((D, H1), lambda i: (0, 0)),
            pl.BlockSpec((1, H1), lambda i: (0, 0)),
            pl.BlockSpec((H1, H2), lambda i: (0, 0)),
            pl.BlockSpec((1, H2), lambda i: (0, 0)),
            pl.BlockSpec((1, H2), lambda i: (0, 0)),
            pl.BlockSpec((1, 1), lambda i: (0, 0)),
        ],
        out_specs=[
            pl.BlockSpec((T, D), lambda i: (i, 0)),
            pl.BlockSpec((S, 128), lambda i: (0, 0)),
        ],
        out_shape=[
            jax.ShapeDtypeStruct((N, D), jnp.float32),
            jax.ShapeDtypeStruct((S, 128), jnp.float32),
        ],
    )(x, seg.reshape(N // T, 1, T), W1, b1.reshape(1, H1), W2,
      b2.reshape(1, H2), W3.reshape(1, H2), b3.reshape(1, 1))


def _seg_body(y_hbm, seg_hbm, pn_hbm, xbuf, segbuf, acc):
    c = lax.axis_index("c")
    s = lax.axis_index("s")
    base = s * RW
    nfull = lax.select(s == 15, (N - 15 * RW) // CH, RW // CH)
    ntail = lax.select(s == 15, (N - 15 * RW - ((N - 15 * RW) // CH) * CH) // 8,
                       0)
    col0 = c * CW
    iota = lax.broadcasted_iota(jnp.int32, (16,), 0)

    def zbody(k, carry):
        acc[k // (CW // 16), pl.ds((k % (CW // 16)) * 16, 16)] = jnp.zeros(
            (16,), jnp.float32)
        return carry

    lax.fori_loop(0, S * CW // 16, zbody, 0)

    NJ = CW // 16
    zero16 = jnp.zeros((16,), jnp.float32)

    def row_block(nrows):
        def seg_at(r):
            rvec = jnp.full((16,), r, jnp.int32)
            return jnp.max(plsc.load_gather(segbuf, [rvec]))

        def flush(prev, cregs):
            for j in range(NJ):
                plsc.addupdate(acc.at[prev, pl.ds(16 * j, 16)], cregs[j])
            return (zero16,) * NJ

        def rbody(r, carry):
            prev, cregs = carry
            seg_r = seg_at(r)
            cregs = lax.cond(seg_r != prev,
                             lambda cr: flush(prev, cr),
                             lambda cr: cr, cregs)
            cregs = tuple(cregs[j] + xbuf[r, pl.ds(16 * j, 16)]
                          for j in range(NJ))
            return (seg_r, cregs)

        def gbody(gi, carry):
            prev, cregs = carry
            r0 = gi * 16
            va = segbuf[pl.ds(r0, 16)]
            uniform = (jnp.max(va) == prev) & (jnp.min(va) == prev)

            def fast(carry_in):
                _, cr = carry_in
                for rr in range(16):
                    cr = tuple(cr[j] + xbuf[r0 + rr, pl.ds(16 * j, 16)]
                               for j in range(NJ))
                return (prev, cr)

            def slow(carry_in):
                return lax.fori_loop(r0, r0 + 16, rbody, carry_in)

            return lax.cond(uniform, fast, slow, (prev, cregs))

        prev0 = seg_at(0)
        init = (prev0, (zero16,) * NJ)
        if nrows % 16 == 0:
            prev, cregs = lax.fori_loop(0, nrows // 16, gbody, init)
        else:
            prev, cregs = lax.fori_loop(0, nrows, rbody, init)
        flush(prev, cregs)

    def body(i, carry):
        off = base + i * CH
        pltpu.sync_copy(y_hbm.at[pl.ds(off, CH), pl.ds(col0, CW)], xbuf)
        pltpu.sync_copy(seg_hbm.at[pl.ds(off, CH)], segbuf)
        row_block(CH)
        return carry

    lax.fori_loop(0, nfull, body, 0)

    def tbody(i, carry):
        off = base + nfull * CH + i * 8
        pltpu.sync_copy(y_hbm.at[pl.ds(off, 8), pl.ds(col0, CW)],
                        xbuf.at[pl.ds(0, 8)])
        pltpu.sync_copy(seg_hbm.at[pl.ds(off, 8)], segbuf.at[pl.ds(0, 8)])
        row_block(8)
        return carry

    lax.fori_loop(0, ntail, tbody, 0)

    pltpu.sync_copy(acc, pn_hbm.at[s, :, pl.ds(col0, CW)])


def _seg_call(y, seg):
    mesh = plsc.VectorSubcoreMesh(core_axis_name="c", subcore_axis_name="s")
    f = pl.kernel(
        _seg_body,
        out_type=jax.ShapeDtypeStruct((16, S, D), jnp.float32),
        mesh=mesh,
        compiler_params=pltpu.CompilerParams(needs_layout_passes=False),
        scratch_types=[
            pltpu.VMEM((CH, CW), jnp.float32),
            pltpu.VMEM((CH,), jnp.int32),
            pltpu.VMEM((S, CW), jnp.float32),
        ],
    )
    return f(y, seg)


def _fin_body(pn_ref, den_ref, o_ref):
    i = pl.program_id(0)

    @pl.when(i == 0)
    def _init():
        o_ref[...] = jnp.zeros_like(o_ref)

    o_ref[...] += pn_ref[0]

    @pl.when(i == 15)
    def _done():
        o_ref[...] = o_ref[...] / den_ref[:, 0:1]


def _fin_call(pn, den):
    return pl.pallas_call(
        _fin_body,
        grid=(16,),
        in_specs=[
            pl.BlockSpec((1, S, D), lambda i: (i, 0, 0)),
            pl.BlockSpec((S, 128), lambda i: (0, 0)),
        ],
        out_specs=pl.BlockSpec((S, D), lambda i: (0, 0)),
        out_shape=jax.ShapeDtypeStruct((S, D), jnp.float32),
    )(pn, den)


def kernel(x, batch_index, W1, b1, W2, b2, W3, b3):
    seg = batch_index.astype(jnp.int32)
    y, den = _mlp_call(x, seg, W1, b1, W2, b2, W3, b3)
    pn = _seg_call(y, seg)
    return _fin_call(pn, den)


# speedup vs baseline: 1.4143x; 1.2084x over previous
import functools

import jax
import jax.numpy as jnp
from jax import lax
from jax.experimental import pallas as pl
from jax.experimental.pallas import tpu as pltpu
from jax.experimental.pallas import tpu_sc as plsc

N = 100000
D = 512
S = 256
H1, H2 = 256, 128

T = 2000

RW = 6272
CH = 112
CW = 256


def _mlp_body(x_ref, seg_ref, w1_ref, b1_ref, w2_ref, b2_ref, w3_ref, b3_ref,
              y_ref, den_ref):
    i = pl.program_id(0)
    x = x_ref[...]
    h = jnp.maximum(
        jnp.dot(x, w1_ref[...], preferred_element_type=jnp.float32)
        + b1_ref[...], 0.0)
    h = jnp.maximum(
        jnp.dot(h, w2_ref[...], preferred_element_type=jnp.float32)
        + b2_ref[...], 0.0)
    logit = jnp.sum(h * w3_ref[...], axis=1, keepdims=True) + b3_ref[...]
    a = jnp.exp(logit)
    y_ref[...] = x * a

    seg = seg_ref[0, 0, :]
    onehot = (seg[None, :] == lax.broadcasted_iota(jnp.int32, (S, T), 0)
              ).astype(jnp.float32)
    a_b = jnp.broadcast_to(a, (a.shape[0], 128))

    @pl.when(i == 0)
    def _init():
        den_ref[...] = jnp.zeros_like(den_ref)

    den_ref[...] += jnp.dot(onehot, a_b, preferred_element_type=jnp.float32)


def _mlp_call(x, seg, W1, b1, W2, b2, W3, b3):
    grid = (N // T,)
    return pl.pallas_call(
        _mlp_body,
        grid=grid,
        in_specs=[
            pl.BlockSpec((T, D), lambda i: (i, 0)),
            pl.BlockSpec((1, 1, T), lambda i: (i, 0, 0)),
            pl.BlockSpec((D, H1), lambda i: (0, 0)),
            pl.BlockSpec((1, H1), lambda i: (0, 0)),
            pl.BlockSpec((H1, H2), lambda i: (0, 0)),
            pl.BlockSpec((1, H2), lambda i: (0, 0)),
            pl.BlockSpec((1, H2), lambda i: (0, 0)),
            pl.BlockSpec((1, 1), lambda i: (0, 0)),
        ],
        out_specs=[
            pl.BlockSpec((T, D), lambda i: (i, 0)),
            pl.BlockSpec((S, 128), lambda i: (0, 0)),
        ],
        out_shape=[
            jax.ShapeDtypeStruct((N, D), jnp.float32),
            jax.ShapeDtypeStruct((S, 128), jnp.float32),
        ],
    )(x, seg.reshape(N // T, 1, T), W1, b1.reshape(1, H1), W2,
      b2.reshape(1, H2), W3.reshape(1, H2), b3.reshape(1, 1))


def _seg_body(y_hbm, seg_hbm, pn_hbm, xbuf0, xbuf1, segbuf0, segbuf1, acc,
              semx0, semx1, sems0, sems1):
    c = lax.axis_index("c")
    s = lax.axis_index("s")
    base = s * RW
    nfull = lax.select(s == 15, (N - 15 * RW) // CH, RW // CH)
    ntail = lax.select(s == 15, (N - 15 * RW - ((N - 15 * RW) // CH) * CH) // 8,
                       0)
    col0 = c * CW

    def zbody(k, carry):
        acc[k // (CW // 16), pl.ds((k % (CW // 16)) * 16, 16)] = jnp.zeros(
            (16,), jnp.float32)
        return carry

    lax.fori_loop(0, S * CW // 16, zbody, 0)

    NJ = CW // 16
    zero16 = jnp.zeros((16,), jnp.float32)

    def row_block(xbuf, segbuf, nrows):
        def seg_at(r):
            rvec = jnp.full((16,), r, jnp.int32)
            return jnp.max(plsc.load_gather(segbuf, [rvec]))

        def flush(prev, cregs):
            for j in range(NJ):
                plsc.addupdate(acc.at[prev, pl.ds(16 * j, 16)], cregs[j])
            return (zero16,) * NJ

        def rbody(r, carry):
            prev, cregs = carry
            seg_r = seg_at(r)
            cregs = lax.cond(seg_r != prev,
                             lambda cr: flush(prev, cr),
                             lambda cr: cr, cregs)
            cregs = tuple(cregs[j] + xbuf[r, pl.ds(16 * j, 16)]
                          for j in range(NJ))
            return (seg_r, cregs)

        def gbody(gi, carry):
            prev, cregs = carry
            r0 = gi * 16
            va = segbuf[pl.ds(r0, 16)]
            uniform = (jnp.max(va) == prev) & (jnp.min(va) == prev)

            def fast(carry_in):
                _, cr = carry_in
                for rr in range(16):
                    cr = tuple(cr[j] + xbuf[r0 + rr, pl.ds(16 * j, 16)]
                               for j in range(NJ))
                return (prev, cr)

            def slow(carry_in):
                return lax.fori_loop(r0, r0 + 16, rbody, carry_in)

            return lax.cond(uniform, fast, slow, (prev, cregs))

        prev0 = seg_at(0)
        init = (prev0, (zero16,) * NJ)
        if nrows % 16 == 0:
            prev, cregs = lax.fori_loop(0, nrows // 16, gbody, init)
        else:
            prev, cregs = lax.fori_loop(0, nrows, rbody, init)
        flush(prev, cregs)

    bufs = ((xbuf0, segbuf0, semx0, sems0), (xbuf1, segbuf1, semx1, sems1))

    def start(idx, xb, sb, sx, ss):
        off = base + idx * CH
        pltpu.async_copy(y_hbm.at[pl.ds(off, CH), pl.ds(col0, CW)], xb, sx)
        pltpu.async_copy(seg_hbm.at[pl.ds(off, CH)], sb, ss)

    def wait(idx, xb, sb, sx, ss):
        off = base + idx * CH
        pltpu.make_async_copy(
            y_hbm.at[pl.ds(off, CH), pl.ds(col0, CW)], xb, sx).wait()
        pltpu.make_async_copy(seg_hbm.at[pl.ds(off, CH)], sb, ss).wait()

    @pl.when(nfull > 0)
    def _p0():
        start(0, *bufs[0])

    @pl.when(nfull > 1)
    def _p1():
        start(1, *bufs[1])

    def body2(i2, carry):
        for b in range(2):
            idx = i2 * 2 + b
            xb, sb, sx, ss = bufs[b]

            @pl.when(idx < nfull)
            def _do():
                wait(idx, xb, sb, sx, ss)
                row_block(xb, sb, CH)

                @pl.when(idx + 2 < nfull)
                def _nxt():
                    start(idx + 2, xb, sb, sx, ss)

        return carry

    lax.fori_loop(0, (nfull + 1) // 2, body2, 0)

    def tbody(i, carry):
        off = base + nfull * CH + i * 8
        pltpu.sync_copy(y_hbm.at[pl.ds(off, 8), pl.ds(col0, CW)],
                        xbuf0.at[pl.ds(0, 8)])
        pltpu.sync_copy(seg_hbm.at[pl.ds(off, 8)], segbuf0.at[pl.ds(0, 8)])
        row_block(xbuf0, segbuf0, 8)
        return carry

    lax.fori_loop(0, ntail, tbody, 0)

    pltpu.sync_copy(acc, pn_hbm.at[s, :, pl.ds(col0, CW)])


def _seg_call(y, seg):
    mesh = plsc.VectorSubcoreMesh(core_axis_name="c", subcore_axis_name="s")
    f = pl.kernel(
        _seg_body,
        out_type=jax.ShapeDtypeStruct((16, S, D), jnp.float32),
        mesh=mesh,
        compiler_params=pltpu.CompilerParams(needs_layout_passes=False),
        scratch_types=[
            pltpu.VMEM((CH, CW), jnp.float32),
            pltpu.VMEM((CH, CW), jnp.float32),
            pltpu.VMEM((CH,), jnp.int32),
            pltpu.VMEM((CH,), jnp.int32),
            pltpu.VMEM((S, CW), jnp.float32),
            pltpu.SemaphoreType.DMA,
            pltpu.SemaphoreType.DMA,
            pltpu.SemaphoreType.DMA,
            pltpu.SemaphoreType.DMA,
        ],
    )
    return f(y, seg)


def _fin_body(pn_ref, den_ref, o_ref):
    i = pl.program_id(0)

    @pl.when(i == 0)
    def _init():
        o_ref[...] = jnp.zeros_like(o_ref)

    o_ref[...] += pn_ref[0]

    @pl.when(i == 15)
    def _done():
        o_ref[...] = o_ref[...] / den_ref[:, 0:1]


def _fin_call(pn, den):
    return pl.pallas_call(
        _fin_body,
        grid=(16,),
        in_specs=[
            pl.BlockSpec((1, S, D), lambda i: (i, 0, 0)),
            pl.BlockSpec((S, 128), lambda i: (0, 0)),
        ],
        out_specs=pl.BlockSpec((S, D), lambda i: (0, 0)),
        out_shape=jax.ShapeDtypeStruct((S, D), jnp.float32),
    )(pn, den)


def kernel(x, batch_index, W1, b1, W2, b2, W3, b3):
    seg = batch_index.astype(jnp.int32)
    y, den = _mlp_call(x, seg, W1, b1, W2, b2, W3, b3)
    pn = _seg_call(y, seg)
    return _fin_call(pn, den)


# speedup vs baseline: 1.7143x; 1.2120x over previous
import functools

import jax
import jax.numpy as jnp
from jax import lax
from jax.experimental import pallas as pl
from jax.experimental.pallas import tpu as pltpu
from jax.experimental.pallas import tpu_sc as plsc

N = 100000
D = 512
S = 256
H1, H2 = 256, 128

T = 2000

K = 2
NH = N // K
RW = 3136
CH = 112
CW = 256


def _mlp_body(x_ref, seg_ref, w1_ref, b1_ref, w2_ref, b2_ref, w3_ref, b3_ref,
              y_ref, den_ref):
    i = pl.program_id(0)
    x = x_ref[...]
    h = jnp.maximum(
        jnp.dot(x, w1_ref[...], preferred_element_type=jnp.float32)
        + b1_ref[...], 0.0)
    h = jnp.maximum(
        jnp.dot(h, w2_ref[...], preferred_element_type=jnp.float32)
        + b2_ref[...], 0.0)
    logit = jnp.sum(h * w3_ref[...], axis=1, keepdims=True) + b3_ref[...]
    a = jnp.exp(logit)
    y_ref[...] = x * a

    seg = seg_ref[0, 0, :]
    onehot = (seg[None, :] == lax.broadcasted_iota(jnp.int32, (S, T), 0)
              ).astype(jnp.float32)
    a_b = jnp.broadcast_to(a, (a.shape[0], 128))

    @pl.when(i == 0)
    def _init():
        den_ref[...] = jnp.zeros_like(den_ref)

    den_ref[...] += jnp.dot(onehot, a_b, preferred_element_type=jnp.float32)


def _mlp_call(x, seg, W1, b1, W2, b2, W3, b3, k):
    grid = (NH // T,)
    off = k * (NH // T)
    return pl.pallas_call(
        _mlp_body,
        grid=grid,
        in_specs=[
            pl.BlockSpec((T, D), lambda i: (i + off, 0)),
            pl.BlockSpec((1, 1, T), lambda i: (i + off, 0, 0)),
            pl.BlockSpec((D, H1), lambda i: (0, 0)),
            pl.BlockSpec((1, H1), lambda i: (0, 0)),
            pl.BlockSpec((H1, H2), lambda i: (0, 0)),
            pl.BlockSpec((1, H2), lambda i: (0, 0)),
            pl.BlockSpec((1, H2), lambda i: (0, 0)),
            pl.BlockSpec((1, 1), lambda i: (0, 0)),
        ],
        out_specs=[
            pl.BlockSpec((T, D), lambda i: (i, 0)),
            pl.BlockSpec((S, 128), lambda i: (0, 0)),
        ],
        out_shape=[
            jax.ShapeDtypeStruct((NH, D), jnp.float32),
            jax.ShapeDtypeStruct((S, 128), jnp.float32),
        ],
    )(x, seg.reshape(N // T, 1, T), W1, b1.reshape(1, H1), W2,
      b2.reshape(1, H2), W3.reshape(1, H2), b3.reshape(1, 1))


def _seg_body(kbase, y_hbm, seg_hbm, pn_hbm, xbuf0, xbuf1, segbuf0, segbuf1,
              acc, semx0, semx1, sems0, sems1):
    c = lax.axis_index("c")
    s = lax.axis_index("s")
    base = s * RW
    nfull = lax.select(s == 15, (NH - 15 * RW) // CH, RW // CH)
    ntail = lax.select(
        s == 15, (NH - 15 * RW - ((NH - 15 * RW) // CH) * CH) // 8, 0)
    col0 = c * CW

    def zbody(k, carry):
        acc[k // (CW // 16), pl.ds((k % (CW // 16)) * 16, 16)] = jnp.zeros(
            (16,), jnp.float32)
        return carry

    lax.fori_loop(0, S * CW // 16, zbody, 0)

    NJ = CW // 16
    zero16 = jnp.zeros((16,), jnp.float32)

    def row_block(xbuf, segbuf, nrows):
        def seg_at(r):
            rvec = jnp.full((16,), r, jnp.int32)
            return jnp.max(plsc.load_gather(segbuf, [rvec]))

        def flush(prev, cregs):
            for j in range(NJ):
                plsc.addupdate(acc.at[prev, pl.ds(16 * j, 16)], cregs[j])
            return (zero16,) * NJ

        def rbody(r, carry):
            prev, cregs = carry
            seg_r = seg_at(r)
            cregs = lax.cond(seg_r != prev,
                             lambda cr: flush(prev, cr),
                             lambda cr: cr, cregs)
            cregs = tuple(cregs[j] + xbuf[r, pl.ds(16 * j, 16)]
                          for j in range(NJ))
            return (seg_r, cregs)

        def gbody(gi, carry):
            prev, cregs = carry
            r0 = gi * 16
            va = segbuf[pl.ds(r0, 16)]
            uniform = (jnp.max(va) == prev) & (jnp.min(va) == prev)

            def fast(carry_in):
                _, cr = carry_in
                for rr in range(16):
                    cr = tuple(cr[j] + xbuf[r0 + rr, pl.ds(16 * j, 16)]
                               for j in range(NJ))
                return (prev, cr)

            def slow(carry_in):
                return lax.fori_loop(r0, r0 + 16, rbody, carry_in)

            return lax.cond(uniform, fast, slow, (prev, cregs))

        prev0 = seg_at(0)
        init = (prev0, (zero16,) * NJ)
        if nrows % 16 == 0:
            prev, cregs = lax.fori_loop(0, nrows // 16, gbody, init)
        else:
            prev, cregs = lax.fori_loop(0, nrows, rbody, init)
        flush(prev, cregs)

    bufs = ((xbuf0, segbuf0, semx0, sems0), (xbuf1, segbuf1, semx1, sems1))

    def start(idx, xb, sb, sx, ss):
        off = base + idx * CH
        pltpu.async_copy(y_hbm.at[pl.ds(off, CH), pl.ds(col0, CW)], xb, sx)
        pltpu.async_copy(seg_hbm.at[pl.ds(kbase + off, CH)], sb, ss)

    def wait(idx, xb, sb, sx, ss):
        off = base + idx * CH
        pltpu.make_async_copy(
            y_hbm.at[pl.ds(off, CH), pl.ds(col0, CW)], xb, sx).wait()
        pltpu.make_async_copy(
            seg_hbm.at[pl.ds(kbase + off, CH)], sb, ss).wait()

    @pl.when(nfull > 0)
    def _p0():
        start(0, *bufs[0])

    @pl.when(nfull > 1)
    def _p1():
        start(1, *bufs[1])

    def body2(i2, carry):
        for b in range(2):
            idx = i2 * 2 + b
            xb, sb, sx, ss = bufs[b]

            @pl.when(idx < nfull)
            def _do():
                wait(idx, xb, sb, sx, ss)
                row_block(xb, sb, CH)

                @pl.when(idx + 2 < nfull)
                def _nxt():
                    start(idx + 2, xb, sb, sx, ss)

        return carry

    lax.fori_loop(0, (nfull + 1) // 2, body2, 0)

    def tbody(i, carry):
        off = base + nfull * CH + i * 8
        pltpu.sync_copy(y_hbm.at[pl.ds(off, 8), pl.ds(col0, CW)],
                        xbuf0.at[pl.ds(0, 8)])
        pltpu.sync_copy(seg_hbm.at[pl.ds(kbase + off, 8)],
                        segbuf0.at[pl.ds(0, 8)])
        row_block(xbuf0, segbuf0, 8)
        return carry

    lax.fori_loop(0, ntail, tbody, 0)

    pltpu.sync_copy(acc, pn_hbm.at[s, :, pl.ds(col0, CW)])


def _seg_call(y, seg, k):
    mesh = plsc.VectorSubcoreMesh(core_axis_name="c", subcore_axis_name="s")
    f = pl.kernel(
        functools.partial(_seg_body, k * NH),
        out_type=jax.ShapeDtypeStruct((16, S, D), jnp.float32),
        mesh=mesh,
        compiler_params=pltpu.CompilerParams(needs_layout_passes=False),
        scratch_types=[
            pltpu.VMEM((CH, CW), jnp.float32),
            pltpu.VMEM((CH, CW), jnp.float32),
            pltpu.VMEM((CH,), jnp.int32),
            pltpu.VMEM((CH,), jnp.int32),
            pltpu.VMEM((S, CW), jnp.float32),
            pltpu.SemaphoreType.DMA,
            pltpu.SemaphoreType.DMA,
            pltpu.SemaphoreType.DMA,
            pltpu.SemaphoreType.DMA,
        ],
    )
    return f(y, seg)


def _fin_body(pn1_ref, pn2_ref, den1_ref, den2_ref, o_ref):
    i = pl.program_id(0)

    @pl.when(i == 0)
    def _init():
        o_ref[...] = jnp.zeros_like(o_ref)

    o_ref[...] += pn1_ref[0] + pn2_ref[0]

    @pl.when(i == 15)
    def _done():
        o_ref[...] = o_ref[...] / (den1_ref[:, 0:1] + den2_ref[:, 0:1])


def _fin_call(pn1, pn2, den1, den2):
    return pl.pallas_call(
        _fin_body,
        grid=(16,),
        in_specs=[
            pl.BlockSpec((1, S, D), lambda i: (i, 0, 0)),
            pl.BlockSpec((1, S, D), lambda i: (i, 0, 0)),
            pl.BlockSpec((S, 128), lambda i: (0, 0)),
            pl.BlockSpec((S, 128), lambda i: (0, 0)),
        ],
        out_specs=pl.BlockSpec((S, D), lambda i: (0, 0)),
        out_shape=jax.ShapeDtypeStruct((S, D), jnp.float32),
    )(pn1, pn2, den1, den2)


def kernel(x, batch_index, W1, b1, W2, b2, W3, b3):
    seg = batch_index.astype(jnp.int32)
    y1, den1 = _mlp_call(x, seg, W1, b1, W2, b2, W3, b3, 0)
    pn1 = _seg_call(y1, seg, 0)
    y2, den2 = _mlp_call(x, seg, W1, b1, W2, b2, W3, b3, 1)
    pn2 = _seg_call(y2, seg, 1)
    return _fin_call(pn1, pn2, den1, den2)
